# Initial kernel scaffold; baseline (speedup 1.0000x reference)
import jax
import jax.numpy as jnp
from jax.experimental import pallas as pl


def kernel(x_product, edge_pb, edge_pc, edge_ps, edge_up, user_emb, brand_emb, category_emb, shop_emb, Wp, bp, W1l, W1r, b1, W2l, W2r, b2):
    raise NotImplementedError("write your pallas kernel here")


# speedup vs baseline: 6.3641x; 6.3641x over previous
import functools

import jax
import jax.numpy as jnp
from jax import lax
from jax.experimental import pallas as pl
from jax.experimental.pallas import tpu as pltpu
from jax.experimental.pallas import tpu_sc as plsc

NP_, NU_, NB_, NC_, NS_ = 50000, 20000, 2000, 500, 5000
N_ = NP_ + NU_ + NB_ + NC_ + NS_
H_, OUT_, DIN_ = 64, 32, 384

E_SIDE = 471040
EPT = E_SIDE // 16
CH = 128
NCH = EPT // CH
ACC_ROWS = 50176
TRASH = 50000
ROWS_PT = ACC_ROWS // 16
ZROWS = 448
OUT_ROWS = 77504

_MESH = plsc.VectorSubcoreMesh(
    core_axis_name="c", subcore_axis_name="s", num_cores=2, num_subcores=16)


def _zero_zbuf(zbuf, width):
    def zrow(i, _):
        for j in range(width // 16):
            zbuf[i, pl.ds(j * 16, 16)] = jnp.zeros((16,), jnp.float32)
        return 0
    lax.fori_loop(0, ZROWS, zrow, 0)


def _zero_acc(acc, zbuf, s):
    def zblk(k, _):
        pltpu.sync_copy(zbuf, acc.at[pl.ds(s * ROWS_PT + k * ZROWS, ZROWS)])
        return 0
    lax.fori_loop(0, ACC_ROWS // 16 // ZROWS, zblk, 0)


def _drain(acc, out, c, s):
    @pl.when((c == 0) & (s < 15))
    def _():
        pltpu.sync_copy(acc.at[pl.ds(s * 3128, 3128)],
                        out.at[pl.ds(s * 3128, 3128)])

    @pl.when((c == 0) & (s == 15))
    def _():
        pltpu.sync_copy(acc.at[pl.ds(15 * 3128, 3080)],
                        out.at[pl.ds(15 * 3128, 3080)])

    @pl.when((c == 1) & (s < 15))
    def _():
        pltpu.sync_copy(acc.at[pl.ds(s * 1720, 1720)],
                        out.at[pl.ds(50000 + s * 1720, 1720)])

    @pl.when((c == 1) & (s == 15))
    def _():
        pltpu.sync_copy(acc.at[pl.ds(15 * 1720, 1704)],
                        out.at[pl.ds(50000 + 15 * 1720, 1704)])


@functools.partial(
    pl.kernel,
    out_type=jax.ShapeDtypeStruct((OUT_ROWS, 32), jnp.float32),
    mesh=_MESH,
    compiler_params=pltpu.CompilerParams(use_tc_tiling_on_sc=False),
    scratch_types=[
        pltpu.VMEM((CH,), jnp.int32),
        pltpu.VMEM((CH,), jnp.int32),
        pltpu.VMEM((CH, 32), jnp.float32),
        pltpu.VMEM_SHARED((ACC_ROWS, 32), jnp.float32),
        pltpu.VMEM((ZROWS, 32), jnp.float32),
        pltpu.SemaphoreType.DMA,
    ],
)
def _sc_segsum(table, srcf, dstf, out, sidx, didx, rows, acc, zbuf, sem):
    c = lax.axis_index("c")
    s = lax.axis_index("s")
    _zero_zbuf(zbuf, 32)
    _zero_acc(acc, zbuf, s)
    plsc.subcore_barrier()
    base = c * E_SIDE + s * EPT

    def chunk(i, _):
        off = pl.multiple_of(base + i * CH, CH)
        pltpu.sync_copy(srcf.at[pl.ds(off, CH)], sidx)
        pltpu.sync_copy(dstf.at[pl.ds(off, CH)], didx)
        pltpu.async_copy(table.at[sidx], rows, sem).wait()
        pltpu.sync_copy(rows, acc.at[didx], add=True)
        return 0

    lax.fori_loop(0, NCH, chunk, 0)
    plsc.subcore_barrier()
    _drain(acc, out, c, s)


@functools.partial(
    pl.kernel,
    out_type=jax.ShapeDtypeStruct((OUT_ROWS, 16), jnp.float32),
    mesh=_MESH,
    compiler_params=pltpu.CompilerParams(use_tc_tiling_on_sc=False),
    scratch_types=[
        pltpu.VMEM((CH,), jnp.int32),
        pltpu.VMEM((CH, 16), jnp.float32),
        pltpu.VMEM_SHARED((ACC_ROWS, 16), jnp.float32),
        pltpu.VMEM((ZROWS, 16), jnp.float32),
    ],
)
def _sc_degree(dstf, out, didx, ones, acc, zbuf):
    c = lax.axis_index("c")
    s = lax.axis_index("s")
    _zero_zbuf(zbuf, 16)
    _zero_acc(acc, zbuf, s)

    def orow(i, _):
        ones[i, pl.ds(0, 16)] = jnp.ones((16,), jnp.float32)
        return 0
    lax.fori_loop(0, CH, orow, 0)
    plsc.subcore_barrier()
    base = c * E_SIDE + s * EPT

    def chunk(i, _):
        off = pl.multiple_of(base + i * CH, CH)
        pltpu.sync_copy(dstf.at[pl.ds(off, CH)], didx)
        pltpu.sync_copy(ones, acc.at[didx], add=True)
        return 0

    lax.fori_loop(0, NCH, chunk, 0)
    plsc.subcore_barrier()
    _drain(acc, out, c, s)


def _k1_body(xp, wp, bp, out):
    out[...] = jax.nn.relu(
        jnp.dot(xp[...], wp[...], preferred_element_type=jnp.float32)
        + bp[...])


def _tc_prod(x_product, Wp, bp):
    R, G = 2000, 25
    return pl.pallas_call(
        _k1_body,
        grid=(G,),
        in_specs=[
            pl.BlockSpec((R, DIN_), lambda i: (i, 0)),
            pl.BlockSpec((DIN_, H_), lambda i: (0, 0)),
            pl.BlockSpec((1, H_), lambda i: (0, 0)),
        ],
        out_specs=pl.BlockSpec((R, H_), lambda i: (i, 0)),
        out_shape=jax.ShapeDtypeStruct((NP_, H_), jnp.float32),
    )(x_product, Wp, bp.reshape(1, H_))


def _k2_body(x, w1l, w1r, b1, outa, outb, outr):
    xb = x[...]
    xl = jnp.dot(xb, w1l[...], preferred_element_type=jnp.float32)
    outa[...] = xl[:, :32]
    outb[...] = xl[:, 32:]
    outr[...] = jnp.dot(xb, w1r[...], preferred_element_type=jnp.float32) \
        + b1[...]


def _tc_lin1(x, W1l, W1r, b1):
    R, G = 3104, 25
    return pl.pallas_call(
        _k2_body,
        grid=(G,),
        in_specs=[
            pl.BlockSpec((R, H_), lambda i: (i, 0)),
            pl.BlockSpec((H_, H_), lambda i: (0, 0)),
            pl.BlockSpec((H_, H_), lambda i: (0, 0)),
            pl.BlockSpec((1, H_), lambda i: (0, 0)),
        ],
        out_specs=[
            pl.BlockSpec((R, 32), lambda i: (i, 0)),
            pl.BlockSpec((R, 32), lambda i: (i, 0)),
            pl.BlockSpec((R, H_), lambda i: (i, 0)),
        ],
        out_shape=[
            jax.ShapeDtypeStruct((N_, 32), jnp.float32),
            jax.ShapeDtypeStruct((N_, 32), jnp.float32),
            jax.ShapeDtypeStruct((N_, H_), jnp.float32),
        ],
    )(x, W1l, W1r, b1.reshape(1, H_))


def _k3_body(agga, aggb, deg, xrb, w2l, w2r, b2, outl, outr):
    inv = 1.0 / jnp.maximum(deg[...][:, :1], 1.0)
    h = jax.nn.relu(
        jnp.concatenate([agga[...] * inv, aggb[...] * inv], axis=1)
        + xrb[...])
    outl[...] = jnp.dot(h, w2l[...], preferred_element_type=jnp.float32)
    outr[...] = jnp.dot(h, w2r[...], preferred_element_type=jnp.float32) \
        + b2[...]


def _tc_layer2in(aggA, aggB, deg16, xrb, W2l, W2r, b2):
    R, G = 3104, 25
    return pl.pallas_call(
        _k3_body,
        grid=(G,),
        in_specs=[
            pl.BlockSpec((R, 32), lambda i: (i, 0)),
            pl.BlockSpec((R, 32), lambda i: (i, 0)),
            pl.BlockSpec((R, 16), lambda i: (i, 0)),
            pl.BlockSpec((R, H_), lambda i: (i, 0)),
            pl.BlockSpec((H_, OUT_), lambda i: (0, 0)),
            pl.BlockSpec((H_, OUT_), lambda i: (0, 0)),
            pl.BlockSpec((1, OUT_), lambda i: (0, 0)),
        ],
        out_specs=[
            pl.BlockSpec((R, OUT_), lambda i: (i, 0)),
            pl.BlockSpec((R, OUT_), lambda i: (i, 0)),
        ],
        out_shape=[
            jax.ShapeDtypeStruct((N_, OUT_), jnp.float32),
            jax.ShapeDtypeStruct((N_, OUT_), jnp.float32),
        ],
    )(aggA, aggB, deg16, xrb, W2l, W2r, b2.reshape(1, OUT_))


def _k4_body(agg2, deg, hrb, out):
    inv = 1.0 / jnp.maximum(deg[...][:, :1], 1.0)
    out[...] = agg2[...] * inv + hrb[...]


def _tc_final(agg2, deg16, hrb2):
    R, G = 3104, 25
    return pl.pallas_call(
        _k4_body,
        grid=(G,),
        in_specs=[
            pl.BlockSpec((R, OUT_), lambda i: (i, 0)),
            pl.BlockSpec((R, 16), lambda i: (i, 0)),
            pl.BlockSpec((R, OUT_), lambda i: (i, 0)),
        ],
        out_specs=pl.BlockSpec((R, OUT_), lambda i: (i, 0)),
        out_shape=jax.ShapeDtypeStruct((N_, OUT_), jnp.float32),
    )(agg2, deg16, hrb2)


def _edges(edge_pb, edge_pc, edge_ps, edge_up):
    i32 = jnp.int32
    npad = E_SIDE - 470000
    padz = jnp.zeros((npad,), i32)
    padt = jnp.full((npad,), TRASH, i32)
    srcA = jnp.concatenate([
        edge_pb[1] + (NP_ + NU_), edge_pc[1] + (NP_ + NU_ + NB_),
        edge_ps[1] + (NP_ + NU_ + NB_ + NC_), edge_up[0] + NP_, padz])
    dstA = jnp.concatenate([
        edge_pb[0], edge_pc[0], edge_ps[0], edge_up[1], padt])
    srcB = jnp.concatenate([
        edge_pb[0], edge_pc[0], edge_ps[0], edge_up[1], padz])
    dstB = jnp.concatenate([
        edge_pb[1] + NU_, edge_pc[1] + (NU_ + NB_),
        edge_ps[1] + (NU_ + NB_ + NC_), edge_up[0], padt])
    return (jnp.concatenate([srcA, srcB]).astype(i32),
            jnp.concatenate([dstA, dstB]).astype(i32))


def kernel(x_product, edge_pb, edge_pc, edge_ps, edge_up, user_emb,
           brand_emb, category_emb, shop_emb, Wp, bp, W1l, W1r, b1,
           W2l, W2r, b2):
    src_flat, dst_flat = _edges(edge_pb, edge_pc, edge_ps, edge_up)
    deg16 = _sc_degree(dst_flat)

    prod = _tc_prod(x_product, Wp, bp)
    x = jnp.concatenate([prod, user_emb, brand_emb, category_emb, shop_emb],
                        axis=0)
    xlA, xlB, xrb = _tc_lin1(x, W1l, W1r, b1)

    aggA = _sc_segsum(xlA, src_flat, dst_flat)
    aggB = _sc_segsum(xlB, src_flat, dst_flat)
    hl, hrb2 = _tc_layer2in(aggA, aggB, deg16, xrb, W2l, W2r, b2)
    agg2 = _sc_segsum(hl, src_flat, dst_flat)
    out = _tc_final(agg2, deg16, hrb2)
    return (out[:NP_], out[NP_:NP_ + NU_], out[NP_ + NU_:NP_ + NU_ + NB_],
            out[NP_ + NU_ + NB_:NP_ + NU_ + NB_ + NC_],
            out[NP_ + NU_ + NB_ + NC_:])


# speedup vs baseline: 9.9798x; 1.5681x over previous
import functools

import jax
import jax.numpy as jnp
from jax import lax
from jax.experimental import pallas as pl
from jax.experimental.pallas import tpu as pltpu
from jax.experimental.pallas import tpu_sc as plsc

NP_, NU_, NB_, NC_, NS_ = 50000, 20000, 2000, 500, 5000
N_ = NP_ + NU_ + NB_ + NC_ + NS_
H_, OUT_, DIN_ = 64, 32, 384

E_REAL = 470000
NMAC = 116
MAC = 256
EPT = NMAC * MAC
E_SIDE = 16 * EPT
ACC_ROWS = 50176
TRASH = 50000
ROWS_PT = ACC_ROWS // 16
ZROWS = 112
OUT_ROWS = 77504
NM_ALL = 2 * 16 * NMAC

_MESH = plsc.VectorSubcoreMesh(
    core_axis_name="c", subcore_axis_name="s", num_cores=2, num_subcores=16)
_SDS = jax.ShapeDtypeStruct


def _zero_zbuf(zbuf):
    def zrow(i, _):
        zbuf[i, pl.ds(0, 16)] = jnp.zeros((16,), jnp.float32)
        zbuf[i, pl.ds(16, 16)] = jnp.zeros((16,), jnp.float32)
        return 0
    lax.fori_loop(0, ZROWS, zrow, 0)


def _zero_acc(acc, zbuf, s):
    def zblk(k, _):
        pltpu.sync_copy(zbuf, acc.at[pl.ds(s * ROWS_PT + k * ZROWS, ZROWS)])
        return 0
    lax.fori_loop(0, ROWS_PT // ZROWS, zblk, 0)


def _drain(acc, out, c, s):
    @pl.when((c == 0) & (s < 15))
    def _():
        pltpu.sync_copy(acc.at[pl.ds(s * 3128, 3128)],
                        out.at[pl.ds(s * 3128, 3128)])

    @pl.when((c == 0) & (s == 15))
    def _():
        pltpu.sync_copy(acc.at[pl.ds(15 * 3128, 3080)],
                        out.at[pl.ds(15 * 3128, 3080)])

    @pl.when((c == 1) & (s < 15))
    def _():
        pltpu.sync_copy(acc.at[pl.ds(s * 1720, 1720)],
                        out.at[pl.ds(50000 + s * 1720, 1720)])

    @pl.when((c == 1) & (s == 15))
    def _():
        pltpu.sync_copy(acc.at[pl.ds(15 * 1720, 1704)],
                        out.at[pl.ds(50000 + 15 * 1720, 1704)])


def _make_segsum():
    scratch = [
        pltpu.VMEM((2, 2, 128), jnp.int32),
        pltpu.VMEM((2, 2, 128), jnp.int32),
        pltpu.VMEM((2, 128, 32), jnp.float32),
        pltpu.VMEM((2, 128, 32), jnp.float32),
        pltpu.VMEM_SHARED((ACC_ROWS, 32), jnp.float32),
        pltpu.VMEM((ZROWS, 32), jnp.float32),
        pltpu.SemaphoreType.DMA,
        pltpu.SemaphoreType.DMA,
    ]

    def body(table, eidx, zd, out, mbufA, mbufB, rowsA, rowsB, acc, zbuf,
             gsem, ssem):
        c = lax.axis_index("c")
        s = lax.axis_index("s")
        _zero_zbuf(zbuf)
        _zero_acc(acc, zbuf, s)
        plsc.subcore_barrier()
        mbase = (c * 16 + s) * NMAC

        def step(g, mbuf, rows, ombuf, orows):
            @pl.when(g >= 2)
            def _():
                pltpu.make_async_copy(zd, rows, ssem).wait()

            pltpu.sync_copy(eidx.at[mbase + g], mbuf)
            for j in range(2):
                pltpu.async_copy(table.at[mbuf.at[0, j]], rows.at[j], gsem)

            @pl.when(g >= 1)
            def _():
                pltpu.make_async_copy(zd, orows, gsem).wait()
                for j in range(2):
                    pltpu.async_copy(orows.at[j], acc.at[ombuf.at[1, j]],
                                     ssem, add=True)

        def chunk(g, _):
            @pl.when(g % 2 == 0)
            def _():
                step(g, mbufA, rowsA, mbufB, rowsB)

            @pl.when(g % 2 == 1)
            def _():
                step(g, mbufB, rowsB, mbufA, rowsA)
            return 0

        lax.fori_loop(0, NMAC, chunk, 0)
        pltpu.make_async_copy(zd, rowsB, gsem).wait()
        for j in range(2):
            pltpu.async_copy(rowsB.at[j], acc.at[mbufB.at[1, j]], ssem,
                             add=True)
        pltpu.make_async_copy(zd, rowsA, ssem).wait()
        pltpu.make_async_copy(zd, rowsB, ssem).wait()
        plsc.subcore_barrier()
        _drain(acc, out, c, s)

    return pl.kernel(
        body,
        out_type=_SDS((OUT_ROWS, 32), jnp.float32),
        mesh=_MESH,
        compiler_params=pltpu.CompilerParams(use_tc_tiling_on_sc=False),
        scratch_types=scratch,
    )


def _make_degree():
    scratch = [
        pltpu.VMEM((2, 128), jnp.int32),
        pltpu.VMEM((2, 128), jnp.int32),
        pltpu.VMEM((2, 128, 16), jnp.float32),
        pltpu.VMEM_SHARED((ACC_ROWS, 16), jnp.float32),
        pltpu.VMEM((ZROWS, 16), jnp.float32),
        pltpu.SemaphoreType.DMA,
    ]

    def body(eidx, zd16, out, dbufA, dbufB, ones16, acc, zbuf16, dsem):
        c = lax.axis_index("c")
        s = lax.axis_index("s")

        def zrow(i, _):
            zbuf16[i, pl.ds(0, 16)] = jnp.zeros((16,), jnp.float32)
            return 0
        lax.fori_loop(0, ZROWS, zrow, 0)

        def orow(i, _):
            ones16[i // 128, i % 128, pl.ds(0, 16)] = \
                jnp.ones((16,), jnp.float32)
            return 0
        lax.fori_loop(0, 256, orow, 0)
        _zero_acc(acc, zbuf16, s)
        plsc.subcore_barrier()
        mbase = (c * 16 + s) * NMAC

        def step(g, dbuf):
            @pl.when(g >= 2)
            def _():
                pltpu.make_async_copy(zd16, ones16, dsem).wait()

            pltpu.sync_copy(eidx.at[mbase + g, 1], dbuf)
            for j in range(2):
                pltpu.async_copy(ones16.at[j], acc.at[dbuf.at[j]], dsem,
                                 add=True)

        def chunk(g, _):
            @pl.when(g % 2 == 0)
            def _():
                step(g, dbufA)

            @pl.when(g % 2 == 1)
            def _():
                step(g, dbufB)
            return 0

        lax.fori_loop(0, NMAC, chunk, 0)
        pltpu.make_async_copy(zd16, ones16, dsem).wait()
        pltpu.make_async_copy(zd16, ones16, dsem).wait()
        plsc.subcore_barrier()
        _drain(acc, out, c, s)

    return pl.kernel(
        body,
        out_type=_SDS((OUT_ROWS, 16), jnp.float32),
        mesh=_MESH,
        compiler_params=pltpu.CompilerParams(use_tc_tiling_on_sc=False),
        scratch_types=scratch,
    )


_segsum = _make_segsum()
_sc_degree = _make_degree()


def _k1_body(xp, wp, bp, out):
    out[...] = jax.nn.relu(
        jnp.dot(xp[...], wp[...], preferred_element_type=jnp.float32)
        + bp[...])


def _tc_prod(x_product, Wp, bp):
    R, G = 2000, 25
    return pl.pallas_call(
        _k1_body,
        grid=(G,),
        in_specs=[
            pl.BlockSpec((R, DIN_), lambda i: (i, 0)),
            pl.BlockSpec((DIN_, H_), lambda i: (0, 0)),
            pl.BlockSpec((1, H_), lambda i: (0, 0)),
        ],
        out_specs=pl.BlockSpec((R, H_), lambda i: (i, 0)),
        out_shape=_SDS((NP_, H_), jnp.float32),
    )(x_product, Wp, bp.reshape(1, H_))


def _k2_body(x, w1l, w1r, b1, outa, outb, outr):
    xb = x[...]
    xl = jnp.dot(xb, w1l[...], preferred_element_type=jnp.float32)
    outa[...] = xl[:, :32]
    outb[...] = xl[:, 32:]
    outr[...] = jnp.dot(xb, w1r[...], preferred_element_type=jnp.float32) \
        + b1[...]


def _tc_lin1(x, W1l, W1r, b1):
    R, G = 3104, 25
    return pl.pallas_call(
        _k2_body,
        grid=(G,),
        in_specs=[
            pl.BlockSpec((R, H_), lambda i: (i, 0)),
            pl.BlockSpec((H_, H_), lambda i: (0, 0)),
            pl.BlockSpec((H_, H_), lambda i: (0, 0)),
            pl.BlockSpec((1, H_), lambda i: (0, 0)),
        ],
        out_specs=[
            pl.BlockSpec((R, 32), lambda i: (i, 0)),
            pl.BlockSpec((R, 32), lambda i: (i, 0)),
            pl.BlockSpec((R, H_), lambda i: (i, 0)),
        ],
        out_shape=[
            _SDS((N_, 32), jnp.float32),
            _SDS((N_, 32), jnp.float32),
            _SDS((N_, H_), jnp.float32),
        ],
    )(x, W1l, W1r, b1.reshape(1, H_))


def _k3_body(agga, aggb, deg, xrb, w2l, w2r, b2, outl, outr):
    inv = 1.0 / jnp.maximum(deg[...][:, :1], 1.0)
    h = jax.nn.relu(
        jnp.concatenate([agga[...] * inv, aggb[...] * inv], axis=1)
        + xrb[...])
    outl[...] = jnp.dot(h, w2l[...], preferred_element_type=jnp.float32)
    outr[...] = jnp.dot(h, w2r[...], preferred_element_type=jnp.float32) \
        + b2[...]


def _tc_layer2in(aggA, aggB, deg16, xrb, W2l, W2r, b2):
    R, G = 3104, 25
    return pl.pallas_call(
        _k3_body,
        grid=(G,),
        in_specs=[
            pl.BlockSpec((R, 32), lambda i: (i, 0)),
            pl.BlockSpec((R, 32), lambda i: (i, 0)),
            pl.BlockSpec((R, 16), lambda i: (i, 0)),
            pl.BlockSpec((R, H_), lambda i: (i, 0)),
            pl.BlockSpec((H_, OUT_), lambda i: (0, 0)),
            pl.BlockSpec((H_, OUT_), lambda i: (0, 0)),
            pl.BlockSpec((1, OUT_), lambda i: (0, 0)),
        ],
        out_specs=[
            pl.BlockSpec((R, OUT_), lambda i: (i, 0)),
            pl.BlockSpec((R, OUT_), lambda i: (i, 0)),
        ],
        out_shape=[
            _SDS((N_, OUT_), jnp.float32),
            _SDS((N_, OUT_), jnp.float32),
        ],
    )(aggA, aggB, deg16, xrb, W2l, W2r, b2.reshape(1, OUT_))


def _k4_body(agg2, deg, hrb, out):
    inv = 1.0 / jnp.maximum(deg[...][:, :1], 1.0)
    out[...] = agg2[...] * inv + hrb[...]


def _tc_final(agg2, deg16, hrb2):
    R, G = 3104, 25
    return pl.pallas_call(
        _k4_body,
        grid=(G,),
        in_specs=[
            pl.BlockSpec((R, OUT_), lambda i: (i, 0)),
            pl.BlockSpec((R, 16), lambda i: (i, 0)),
            pl.BlockSpec((R, OUT_), lambda i: (i, 0)),
        ],
        out_specs=pl.BlockSpec((R, OUT_), lambda i: (i, 0)),
        out_shape=_SDS((N_, OUT_), jnp.float32),
    )(agg2, deg16, hrb2)


def _edges(edge_pb, edge_pc, edge_ps, edge_up):
    i32 = jnp.int32
    npad = E_SIDE - E_REAL
    padz = jnp.zeros((npad,), i32)
    padt = jnp.full((npad,), TRASH, i32)
    srcA = jnp.concatenate([
        edge_pb[1] + (NP_ + NU_), edge_pc[1] + (NP_ + NU_ + NB_),
        edge_ps[1] + (NP_ + NU_ + NB_ + NC_), edge_up[0] + NP_, padz])
    dstA = jnp.concatenate([
        edge_pb[0], edge_pc[0], edge_ps[0], edge_up[1], padt])
    srcB = jnp.concatenate([
        edge_pb[0], edge_pc[0], edge_ps[0], edge_up[1], padz])
    dstB = jnp.concatenate([
        edge_pb[1] + NU_, edge_pc[1] + (NU_ + NB_),
        edge_ps[1] + (NU_ + NB_ + NC_), edge_up[0], padt])
    sr = jnp.concatenate([srcA, srcB]).astype(i32).reshape(2, 16, NMAC, 2, 128)
    dr = jnp.concatenate([dstA, dstB]).astype(i32).reshape(2, 16, NMAC, 2, 128)
    return jnp.stack([sr, dr], axis=3).reshape(NM_ALL, 2, 2, 128)


def kernel(x_product, edge_pb, edge_pc, edge_ps, edge_up, user_emb,
           brand_emb, category_emb, shop_emb, Wp, bp, W1l, W1r, b1,
           W2l, W2r, b2):
    eidx = _edges(edge_pb, edge_pc, edge_ps, edge_up)
    zd = jnp.zeros((2, 128, 32), jnp.float32)
    zd16 = jnp.zeros((2, 128, 16), jnp.float32)
    deg16 = _sc_degree(eidx, zd16)

    prod = _tc_prod(x_product, Wp, bp)
    x = jnp.concatenate([prod, user_emb, brand_emb, category_emb, shop_emb],
                        axis=0)
    xlA, xlB, xrb = _tc_lin1(x, W1l, W1r, b1)

    aggA = _segsum(xlA, eidx, zd)
    aggB = _segsum(xlB, eidx, zd)
    hl, hrb2 = _tc_layer2in(aggA, aggB, deg16, xrb, W2l, W2r, b2)
    agg2 = _segsum(hl, eidx, zd)
    out = _tc_final(agg2, deg16, hrb2)
    return (out[:NP_], out[NP_:NP_ + NU_], out[NP_ + NU_:NP_ + NU_ + NB_],
            out[NP_ + NU_ + NB_:NP_ + NU_ + NB_ + NC_],
            out[NP_ + NU_ + NB_ + NC_:])


# speedup vs baseline: 10.5464x; 1.0568x over previous
import functools

import jax
import jax.numpy as jnp
from jax import lax
from jax.experimental import pallas as pl
from jax.experimental.pallas import tpu as pltpu
from jax.experimental.pallas import tpu_sc as plsc

NP_, NU_, NB_, NC_, NS_ = 50000, 20000, 2000, 500, 5000
N_ = NP_ + NU_ + NB_ + NC_ + NS_
H_, OUT_, DIN_ = 64, 32, 384

E_REAL = 470000
NMAC = 116
MAC = 256
EPT = NMAC * MAC
E_SIDE = 16 * EPT
ACC_ROWS = 50176
TRASH = 50000
ROWS_PT = ACC_ROWS // 16
ZROWS = 56
OUT_ROWS = 77504
NM_ALL = 2 * 16 * NMAC

_MESH = plsc.VectorSubcoreMesh(
    core_axis_name="c", subcore_axis_name="s", num_cores=2, num_subcores=16)
_SDS = jax.ShapeDtypeStruct


def _zero_zbuf(zbuf):
    def zrow(i, _):
        zbuf[i, pl.ds(0, 16)] = jnp.zeros((16,), jnp.float32)
        zbuf[i, pl.ds(16, 16)] = jnp.zeros((16,), jnp.float32)
        return 0
    lax.fori_loop(0, ZROWS, zrow, 0)


def _zero_acc(acc, zbuf, s):
    def zblk(k, _):
        pltpu.sync_copy(zbuf, acc.at[pl.ds(s * ROWS_PT + k * ZROWS, ZROWS)])
        return 0
    lax.fori_loop(0, ROWS_PT // ZROWS, zblk, 0)


def _drain(acc, out, c, s):
    @pl.when((c == 0) & (s < 15))
    def _():
        pltpu.sync_copy(acc.at[pl.ds(s * 3128, 3128)],
                        out.at[pl.ds(s * 3128, 3128)])

    @pl.when((c == 0) & (s == 15))
    def _():
        pltpu.sync_copy(acc.at[pl.ds(15 * 3128, 3080)],
                        out.at[pl.ds(15 * 3128, 3080)])

    @pl.when((c == 1) & (s < 15))
    def _():
        pltpu.sync_copy(acc.at[pl.ds(s * 1720, 1720)],
                        out.at[pl.ds(50000 + s * 1720, 1720)])

    @pl.when((c == 1) & (s == 15))
    def _():
        pltpu.sync_copy(acc.at[pl.ds(15 * 1720, 1704)],
                        out.at[pl.ds(50000 + 15 * 1720, 1704)])


def _make_segsum():
    scratch = [
        pltpu.VMEM((2, 2, 128), jnp.int32),
        pltpu.VMEM((2, 2, 128), jnp.int32),
        pltpu.VMEM((2, 2, 128), jnp.int32),
        pltpu.VMEM((2, 128, 32), jnp.float32),
        pltpu.VMEM((2, 128, 32), jnp.float32),
        pltpu.VMEM((2, 128, 32), jnp.float32),
        pltpu.VMEM_SHARED((ACC_ROWS, 32), jnp.float32),
        pltpu.VMEM((ZROWS, 32), jnp.float32),
        pltpu.SemaphoreType.DMA,
        pltpu.SemaphoreType.DMA,
        pltpu.SemaphoreType.DMA,
    ]

    def body(table, eidx, zd, out, mbuf0, mbuf1, mbuf2, rows0, rows1, rows2,
             acc, zbuf, isem, gsem, ssem):
        c = lax.axis_index("c")
        s = lax.axis_index("s")
        mbufs = (mbuf0, mbuf1, mbuf2)
        rows = (rows0, rows1, rows2)
        _zero_zbuf(zbuf)
        _zero_acc(acc, zbuf, s)
        plsc.subcore_barrier()
        mbase = (c * 16 + s) * NMAC
        pltpu.sync_copy(eidx.at[mbase], mbuf0)

        def step(g, p):
            mb, rw = mbufs[p], rows[p]
            omb, orw = mbufs[(p + 2) % 3], rows[(p + 2) % 3]

            @pl.when(g >= 2)
            def _():
                pltpu.make_async_copy(zd, rows[(p + 1) % 3], ssem).wait()

            @pl.when(g >= 1)
            def _():
                pltpu.make_async_copy(eidx.at[0], mb, isem).wait()

            @pl.when(g < NMAC - 1)
            def _():
                pltpu.async_copy(eidx.at[mbase + g + 1],
                                 mbufs[(p + 1) % 3], isem)

            for j in range(2):
                pltpu.async_copy(table.at[mb.at[0, j]], rw.at[j], gsem)

            @pl.when(g >= 1)
            def _():
                pltpu.make_async_copy(zd, orw, gsem).wait()
                for j in range(2):
                    pltpu.async_copy(orw.at[j], acc.at[omb.at[1, j]],
                                     ssem, add=True)

        def chunk(g, _):
            for p in range(3):
                @pl.when(g % 3 == p)
                def _(p=p):
                    step(g, p)
            return 0

        lax.fori_loop(0, NMAC, chunk, 0)
        pltpu.make_async_copy(zd, rows1, gsem).wait()
        for j in range(2):
            pltpu.async_copy(rows1.at[j], acc.at[mbuf1.at[1, j]], ssem,
                             add=True)
        pltpu.make_async_copy(zd, rows0, ssem).wait()
        pltpu.make_async_copy(zd, rows1, ssem).wait()
        plsc.subcore_barrier()
        _drain(acc, out, c, s)

    return pl.kernel(
        body,
        out_type=_SDS((OUT_ROWS, 32), jnp.float32),
        mesh=_MESH,
        compiler_params=pltpu.CompilerParams(use_tc_tiling_on_sc=False),
        scratch_types=scratch,
    )


def _make_degree():
    scratch = [
        pltpu.VMEM((2, 128), jnp.int32),
        pltpu.VMEM((2, 128), jnp.int32),
        pltpu.VMEM((2, 128, 16), jnp.float32),
        pltpu.VMEM_SHARED((ACC_ROWS, 16), jnp.float32),
        pltpu.VMEM((ZROWS, 16), jnp.float32),
        pltpu.SemaphoreType.DMA,
    ]

    def body(eidx, zd16, out, dbufA, dbufB, ones16, acc, zbuf16, dsem):
        c = lax.axis_index("c")
        s = lax.axis_index("s")

        def zrow(i, _):
            zbuf16[i, pl.ds(0, 16)] = jnp.zeros((16,), jnp.float32)
            return 0
        lax.fori_loop(0, ZROWS, zrow, 0)

        def orow(i, _):
            ones16[i // 128, i % 128, pl.ds(0, 16)] = \
                jnp.ones((16,), jnp.float32)
            return 0
        lax.fori_loop(0, 256, orow, 0)
        _zero_acc(acc, zbuf16, s)
        plsc.subcore_barrier()
        mbase = (c * 16 + s) * NMAC

        def step(g, dbuf):
            @pl.when(g >= 2)
            def _():
                pltpu.make_async_copy(zd16, ones16, dsem).wait()

            pltpu.sync_copy(eidx.at[mbase + g, 1], dbuf)
            for j in range(2):
                pltpu.async_copy(ones16.at[j], acc.at[dbuf.at[j]], dsem,
                                 add=True)

        def chunk(g, _):
            @pl.when(g % 2 == 0)
            def _():
                step(g, dbufA)

            @pl.when(g % 2 == 1)
            def _():
                step(g, dbufB)
            return 0

        lax.fori_loop(0, NMAC, chunk, 0)
        pltpu.make_async_copy(zd16, ones16, dsem).wait()
        pltpu.make_async_copy(zd16, ones16, dsem).wait()
        plsc.subcore_barrier()
        _drain(acc, out, c, s)

    return pl.kernel(
        body,
        out_type=_SDS((OUT_ROWS, 16), jnp.float32),
        mesh=_MESH,
        compiler_params=pltpu.CompilerParams(use_tc_tiling_on_sc=False),
        scratch_types=scratch,
    )


_segsum = _make_segsum()
_sc_degree = _make_degree()


def _k1_body(xp, wp, bp, out):
    out[...] = jax.nn.relu(
        jnp.dot(xp[...], wp[...], preferred_element_type=jnp.float32)
        + bp[...])


def _tc_prod(x_product, Wp, bp):
    R, G = 2000, 25
    return pl.pallas_call(
        _k1_body,
        grid=(G,),
        in_specs=[
            pl.BlockSpec((R, DIN_), lambda i: (i, 0)),
            pl.BlockSpec((DIN_, H_), lambda i: (0, 0)),
            pl.BlockSpec((1, H_), lambda i: (0, 0)),
        ],
        out_specs=pl.BlockSpec((R, H_), lambda i: (i, 0)),
        out_shape=_SDS((NP_, H_), jnp.float32),
    )(x_product, Wp, bp.reshape(1, H_))


def _k2_body(x, w1l, w1r, b1, outa, outb, outr):
    xb = x[...]
    xl = jnp.dot(xb, w1l[...], preferred_element_type=jnp.float32)
    outa[...] = xl[:, :32]
    outb[...] = xl[:, 32:]
    outr[...] = jnp.dot(xb, w1r[...], preferred_element_type=jnp.float32) \
        + b1[...]


def _tc_lin1(x, W1l, W1r, b1):
    R, G = 3104, 25
    return pl.pallas_call(
        _k2_body,
        grid=(G,),
        in_specs=[
            pl.BlockSpec((R, H_), lambda i: (i, 0)),
            pl.BlockSpec((H_, H_), lambda i: (0, 0)),
            pl.BlockSpec((H_, H_), lambda i: (0, 0)),
            pl.BlockSpec((1, H_), lambda i: (0, 0)),
        ],
        out_specs=[
            pl.BlockSpec((R, 32), lambda i: (i, 0)),
            pl.BlockSpec((R, 32), lambda i: (i, 0)),
            pl.BlockSpec((R, H_), lambda i: (i, 0)),
        ],
        out_shape=[
            _SDS((N_, 32), jnp.float32),
            _SDS((N_, 32), jnp.float32),
            _SDS((N_, H_), jnp.float32),
        ],
    )(x, W1l, W1r, b1.reshape(1, H_))


def _k3_body(agga, aggb, deg, xrb, w2l, w2r, b2, outl, outr):
    inv = 1.0 / jnp.maximum(deg[...][:, :1], 1.0)
    h = jax.nn.relu(
        jnp.concatenate([agga[...] * inv, aggb[...] * inv], axis=1)
        + xrb[...])
    outl[...] = jnp.dot(h, w2l[...], preferred_element_type=jnp.float32)
    outr[...] = jnp.dot(h, w2r[...], preferred_element_type=jnp.float32) \
        + b2[...]


def _tc_layer2in(aggA, aggB, deg16, xrb, W2l, W2r, b2):
    R, G = 3104, 25
    return pl.pallas_call(
        _k3_body,
        grid=(G,),
        in_specs=[
            pl.BlockSpec((R, 32), lambda i: (i, 0)),
            pl.BlockSpec((R, 32), lambda i: (i, 0)),
            pl.BlockSpec((R, 16), lambda i: (i, 0)),
            pl.BlockSpec((R, H_), lambda i: (i, 0)),
            pl.BlockSpec((H_, OUT_), lambda i: (0, 0)),
            pl.BlockSpec((H_, OUT_), lambda i: (0, 0)),
            pl.BlockSpec((1, OUT_), lambda i: (0, 0)),
        ],
        out_specs=[
            pl.BlockSpec((R, OUT_), lambda i: (i, 0)),
            pl.BlockSpec((R, OUT_), lambda i: (i, 0)),
        ],
        out_shape=[
            _SDS((N_, OUT_), jnp.float32),
            _SDS((N_, OUT_), jnp.float32),
        ],
    )(aggA, aggB, deg16, xrb, W2l, W2r, b2.reshape(1, OUT_))


def _k4_body(agg2, deg, hrb, out):
    inv = 1.0 / jnp.maximum(deg[...][:, :1], 1.0)
    out[...] = agg2[...] * inv + hrb[...]


def _tc_final(agg2, deg16, hrb2):
    R, G = 3104, 25
    return pl.pallas_call(
        _k4_body,
        grid=(G,),
        in_specs=[
            pl.BlockSpec((R, OUT_), lambda i: (i, 0)),
            pl.BlockSpec((R, 16), lambda i: (i, 0)),
            pl.BlockSpec((R, OUT_), lambda i: (i, 0)),
        ],
        out_specs=pl.BlockSpec((R, OUT_), lambda i: (i, 0)),
        out_shape=_SDS((N_, OUT_), jnp.float32),
    )(agg2, deg16, hrb2)


def _edges(edge_pb, edge_pc, edge_ps, edge_up):
    i32 = jnp.int32
    npad = E_SIDE - E_REAL
    padz = jnp.zeros((npad,), i32)
    padt = jnp.full((npad,), TRASH, i32)
    srcA = jnp.concatenate([
        edge_pb[1] + (NP_ + NU_), edge_pc[1] + (NP_ + NU_ + NB_),
        edge_ps[1] + (NP_ + NU_ + NB_ + NC_), edge_up[0] + NP_, padz])
    dstA = jnp.concatenate([
        edge_pb[0], edge_pc[0], edge_ps[0], edge_up[1], padt])
    srcB = jnp.concatenate([
        edge_pb[0], edge_pc[0], edge_ps[0], edge_up[1], padz])
    dstB = jnp.concatenate([
        edge_pb[1] + NU_, edge_pc[1] + (NU_ + NB_),
        edge_ps[1] + (NU_ + NB_ + NC_), edge_up[0], padt])
    sr = jnp.concatenate([srcA, srcB]).astype(i32).reshape(2, 16, NMAC, 2, 128)
    dr = jnp.concatenate([dstA, dstB]).astype(i32).reshape(2, 16, NMAC, 2, 128)
    return jnp.stack([sr, dr], axis=3).reshape(NM_ALL, 2, 2, 128)


def kernel(x_product, edge_pb, edge_pc, edge_ps, edge_up, user_emb,
           brand_emb, category_emb, shop_emb, Wp, bp, W1l, W1r, b1,
           W2l, W2r, b2):
    eidx = _edges(edge_pb, edge_pc, edge_ps, edge_up)
    zd = jnp.zeros((2, 128, 32), jnp.float32)
    zd16 = jnp.zeros((2, 128, 16), jnp.float32)
    deg16 = _sc_degree(eidx, zd16)

    prod = _tc_prod(x_product, Wp, bp)
    x = jnp.concatenate([prod, user_emb, brand_emb, category_emb, shop_emb],
                        axis=0)
    xlA, xlB, xrb = _tc_lin1(x, W1l, W1r, b1)

    aggA = _segsum(xlA, eidx, zd)
    aggB = _segsum(xlB, eidx, zd)
    hl, hrb2 = _tc_layer2in(aggA, aggB, deg16, xrb, W2l, W2r, b2)
    agg2 = _segsum(hl, eidx, zd)
    out = _tc_final(agg2, deg16, hrb2)
    return (out[:NP_], out[NP_:NP_ + NU_], out[NP_ + NU_:NP_ + NU_ + NB_],
            out[NP_ + NU_ + NB_:NP_ + NU_ + NB_ + NC_],
            out[NP_ + NU_ + NB_ + NC_:])


# speedup vs baseline: 13.1135x; 1.2434x over previous
import functools

import jax
import jax.numpy as jnp
from jax import lax
from jax.experimental import pallas as pl
from jax.experimental.pallas import tpu as pltpu
from jax.experimental.pallas import tpu_sc as plsc

NP_, NU_, NB_, NC_, NS_ = 50000, 20000, 2000, 500, 5000
N_ = NP_ + NU_ + NB_ + NC_ + NS_
H_, OUT_, DIN_ = 64, 32, 384

E_REAL = 470000
NMAC = 115
MAC = 256
EPT = NMAC * MAC
E_SIDE = 16 * EPT
ACC_ROWS = 50176
TRASH = 50000
ROWS_PT = ACC_ROWS // 16
ZROWS = 56
OUT_ROWS = 77504
NM_ALL = 2 * 16 * NMAC

_MESH = plsc.VectorSubcoreMesh(
    core_axis_name="c", subcore_axis_name="s", num_cores=2, num_subcores=16)
_SDS = jax.ShapeDtypeStruct


def _zero_zbuf(zbuf):
    def zrow(i, _):
        zbuf[i, pl.ds(0, 16)] = jnp.zeros((16,), jnp.float32)
        zbuf[i, pl.ds(16, 16)] = jnp.zeros((16,), jnp.float32)
        return 0
    lax.fori_loop(0, ZROWS, zrow, 0)


def _zero_acc(acc, zbuf, s):
    def zblk(k, _):
        pltpu.sync_copy(zbuf, acc.at[pl.ds(s * ROWS_PT + k * ZROWS, ZROWS)])
        return 0
    lax.fori_loop(0, ROWS_PT // ZROWS, zblk, 0)


def _drain(acc, out, c, s):
    @pl.when((c == 0) & (s < 15))
    def _():
        pltpu.sync_copy(acc.at[pl.ds(s * 3128, 3128)],
                        out.at[pl.ds(s * 3128, 3128)])

    @pl.when((c == 0) & (s == 15))
    def _():
        pltpu.sync_copy(acc.at[pl.ds(15 * 3128, 3080)],
                        out.at[pl.ds(15 * 3128, 3080)])

    @pl.when((c == 1) & (s < 15))
    def _():
        pltpu.sync_copy(acc.at[pl.ds(s * 1720, 1720)],
                        out.at[pl.ds(50000 + s * 1720, 1720)])

    @pl.when((c == 1) & (s == 15))
    def _():
        pltpu.sync_copy(acc.at[pl.ds(15 * 1720, 1704)],
                        out.at[pl.ds(50000 + 15 * 1720, 1704)])


def _make_segsum():
    scratch = [
        pltpu.VMEM((2, 2, 128), jnp.int32),
        pltpu.VMEM((2, 2, 128), jnp.int32),
        pltpu.VMEM((2, 2, 128), jnp.int32),
        pltpu.VMEM((2, 128, 32), jnp.float32),
        pltpu.VMEM((2, 128, 32), jnp.float32),
        pltpu.VMEM((2, 128, 32), jnp.float32),
        pltpu.VMEM_SHARED((ACC_ROWS, 32), jnp.float32),
        pltpu.VMEM((ZROWS, 32), jnp.float32),
        pltpu.SemaphoreType.DMA,
        pltpu.SemaphoreType.DMA,
        pltpu.SemaphoreType.DMA,
    ]

    def body(table, eidx, zd, out, mbuf0, mbuf1, mbuf2, rows0, rows1, rows2,
             acc, zbuf, isem, gsem, ssem):
        c = lax.axis_index("c")
        s = lax.axis_index("s")
        mbufs = (mbuf0, mbuf1, mbuf2)
        rows = (rows0, rows1, rows2)
        _zero_zbuf(zbuf)
        _zero_acc(acc, zbuf, s)
        plsc.subcore_barrier()
        mbase = (c * 16 + s) * NMAC
        pltpu.sync_copy(eidx.at[mbase], mbuf0)

        def step(g, p):
            mb, rw = mbufs[p], rows[p]
            omb, orw = mbufs[(p + 2) % 3], rows[(p + 2) % 3]

            @pl.when(g >= 2)
            def _():
                pltpu.make_async_copy(zd, rows[(p + 1) % 3], ssem).wait()

            @pl.when(g >= 1)
            def _():
                pltpu.make_async_copy(eidx.at[0], mb, isem).wait()

            @pl.when(g < NMAC - 1)
            def _():
                pltpu.async_copy(eidx.at[mbase + g + 1],
                                 mbufs[(p + 1) % 3], isem)

            for j in range(2):
                pltpu.async_copy(table.at[mb.at[0, j]], rw.at[j], gsem)

            @pl.when(g >= 1)
            def _():
                pltpu.make_async_copy(zd, orw, gsem).wait()
                for j in range(2):
                    pltpu.async_copy(orw.at[j], acc.at[omb.at[1, j]],
                                     ssem, add=True)

        def chunk(g, _):
            for p in range(3):
                @pl.when(g % 3 == p)
                def _(p=p):
                    step(g, p)
            return 0

        lax.fori_loop(0, NMAC, chunk, 0)
        pltpu.make_async_copy(zd, rows0, gsem).wait()
        for j in range(2):
            pltpu.async_copy(rows0.at[j], acc.at[mbuf0.at[1, j]], ssem,
                             add=True)
        pltpu.make_async_copy(zd, rows2, ssem).wait()
        pltpu.make_async_copy(zd, rows0, ssem).wait()
        plsc.subcore_barrier()
        _drain(acc, out, c, s)

    return pl.kernel(
        body,
        out_type=_SDS((OUT_ROWS, 32), jnp.float32),
        mesh=_MESH,
        compiler_params=pltpu.CompilerParams(use_tc_tiling_on_sc=False),
        scratch_types=scratch,
    )


def _make_degree():
    scratch = [
        pltpu.VMEM((2, 128), jnp.int32),
        pltpu.VMEM((2, 128), jnp.int32),
        pltpu.VMEM((2, 128, 16), jnp.float32),
        pltpu.VMEM_SHARED((ACC_ROWS, 16), jnp.float32),
        pltpu.VMEM((ZROWS, 16), jnp.float32),
        pltpu.SemaphoreType.DMA,
    ]

    def body(eidx, zd16, out, dbufA, dbufB, ones16, acc, zbuf16, dsem):
        c = lax.axis_index("c")
        s = lax.axis_index("s")

        def zrow(i, _):
            zbuf16[i, pl.ds(0, 16)] = jnp.zeros((16,), jnp.float32)
            return 0
        lax.fori_loop(0, ZROWS, zrow, 0)

        def orow(i, _):
            ones16[i // 128, i % 128, pl.ds(0, 16)] = \
                jnp.ones((16,), jnp.float32)
            return 0
        lax.fori_loop(0, 256, orow, 0)
        _zero_acc(acc, zbuf16, s)
        plsc.subcore_barrier()
        mbase = (c * 16 + s) * NMAC

        def step(g, dbuf):
            @pl.when(g >= 2)
            def _():
                pltpu.make_async_copy(zd16, ones16, dsem).wait()

            pltpu.sync_copy(eidx.at[mbase + g, 1], dbuf)
            for j in range(2):
                pltpu.async_copy(ones16.at[j], acc.at[dbuf.at[j]], dsem,
                                 add=True)

        def chunk(g, _):
            @pl.when(g % 2 == 0)
            def _():
                step(g, dbufA)

            @pl.when(g % 2 == 1)
            def _():
                step(g, dbufB)
            return 0

        lax.fori_loop(0, NMAC, chunk, 0)
        pltpu.make_async_copy(zd16, ones16, dsem).wait()
        pltpu.make_async_copy(zd16, ones16, dsem).wait()
        plsc.subcore_barrier()
        _drain(acc, out, c, s)

    return pl.kernel(
        body,
        out_type=_SDS((OUT_ROWS, 16), jnp.float32),
        mesh=_MESH,
        compiler_params=pltpu.CompilerParams(use_tc_tiling_on_sc=False),
        scratch_types=scratch,
    )


_segsum = _make_segsum()
_sc_degree = _make_degree()


def _k1_body(xp, wp, bp, out):
    out[...] = jax.nn.relu(
        jnp.dot(xp[...], wp[...], preferred_element_type=jnp.float32)
        + bp[...])


def _tc_prod(x_product, Wp, bp):
    R, G = 2000, 25
    return pl.pallas_call(
        _k1_body,
        grid=(G,),
        in_specs=[
            pl.BlockSpec((R, DIN_), lambda i: (i, 0)),
            pl.BlockSpec((DIN_, H_), lambda i: (0, 0)),
            pl.BlockSpec((1, H_), lambda i: (0, 0)),
        ],
        out_specs=pl.BlockSpec((R, H_), lambda i: (i, 0)),
        out_shape=_SDS((NP_, H_), jnp.float32),
    )(x_product, Wp, bp.reshape(1, H_))


def _k2_body(x, w1l, w1r, b1, outa, outb, outr):
    xb = x[...]
    xl = jnp.dot(xb, w1l[...], preferred_element_type=jnp.float32)
    outa[...] = xl[:, :32]
    outb[...] = xl[:, 32:]
    outr[...] = jnp.dot(xb, w1r[...], preferred_element_type=jnp.float32) \
        + b1[...]


def _tc_lin1(x, W1l, W1r, b1):
    R, G = 3104, 25
    return pl.pallas_call(
        _k2_body,
        grid=(G,),
        in_specs=[
            pl.BlockSpec((R, H_), lambda i: (i, 0)),
            pl.BlockSpec((H_, H_), lambda i: (0, 0)),
            pl.BlockSpec((H_, H_), lambda i: (0, 0)),
            pl.BlockSpec((1, H_), lambda i: (0, 0)),
        ],
        out_specs=[
            pl.BlockSpec((R, 32), lambda i: (i, 0)),
            pl.BlockSpec((R, 32), lambda i: (i, 0)),
            pl.BlockSpec((R, H_), lambda i: (i, 0)),
        ],
        out_shape=[
            _SDS((N_, 32), jnp.float32),
            _SDS((N_, 32), jnp.float32),
            _SDS((N_, H_), jnp.float32),
        ],
    )(x, W1l, W1r, b1.reshape(1, H_))


def _k3_body(agga, aggb, deg, xrb, w2l, w2r, b2, outl, outr):
    inv = 1.0 / jnp.maximum(deg[...][:, :1], 1.0)
    h = jax.nn.relu(
        jnp.concatenate([agga[...] * inv, aggb[...] * inv], axis=1)
        + xrb[...])
    outl[...] = jnp.dot(h, w2l[...], preferred_element_type=jnp.float32)
    outr[...] = jnp.dot(h, w2r[...], preferred_element_type=jnp.float32) \
        + b2[...]


def _tc_layer2in(aggA, aggB, deg16, xrb, W2l, W2r, b2):
    R, G = 3104, 25
    return pl.pallas_call(
        _k3_body,
        grid=(G,),
        in_specs=[
            pl.BlockSpec((R, 32), lambda i: (i, 0)),
            pl.BlockSpec((R, 32), lambda i: (i, 0)),
            pl.BlockSpec((R, 16), lambda i: (i, 0)),
            pl.BlockSpec((R, H_), lambda i: (i, 0)),
            pl.BlockSpec((H_, OUT_), lambda i: (0, 0)),
            pl.BlockSpec((H_, OUT_), lambda i: (0, 0)),
            pl.BlockSpec((1, OUT_), lambda i: (0, 0)),
        ],
        out_specs=[
            pl.BlockSpec((R, OUT_), lambda i: (i, 0)),
            pl.BlockSpec((R, OUT_), lambda i: (i, 0)),
        ],
        out_shape=[
            _SDS((N_, OUT_), jnp.float32),
            _SDS((N_, OUT_), jnp.float32),
        ],
    )(aggA, aggB, deg16, xrb, W2l, W2r, b2.reshape(1, OUT_))


def _k4_body(agg2, deg, hrb, out):
    inv = 1.0 / jnp.maximum(deg[...][:, :1], 1.0)
    out[...] = agg2[...] * inv + hrb[...]


def _tc_final(agg2, deg16, hrb2):
    R, G = 3104, 25
    return pl.pallas_call(
        _k4_body,
        grid=(G,),
        in_specs=[
            pl.BlockSpec((R, OUT_), lambda i: (i, 0)),
            pl.BlockSpec((R, 16), lambda i: (i, 0)),
            pl.BlockSpec((R, OUT_), lambda i: (i, 0)),
        ],
        out_specs=pl.BlockSpec((R, OUT_), lambda i: (i, 0)),
        out_shape=_SDS((N_, OUT_), jnp.float32),
    )(agg2, deg16, hrb2)


def _edges(edge_pb, edge_pc, edge_ps, edge_up):
    i32 = jnp.int32
    npad = E_SIDE - E_REAL
    padz = jnp.zeros((npad,), i32)
    padt = jnp.full((npad,), TRASH, i32)
    srcA = jnp.concatenate([
        edge_pb[1] + (NP_ + NU_), edge_pc[1] + (NP_ + NU_ + NB_),
        edge_ps[1] + (NP_ + NU_ + NB_ + NC_), edge_up[0] + NP_, padz])
    dstA = jnp.concatenate([
        edge_pb[0], edge_pc[0], edge_ps[0], edge_up[1], padt])
    srcB = jnp.concatenate([
        edge_pb[0], edge_pc[0], edge_ps[0], edge_up[1], padz])
    dstB = jnp.concatenate([
        edge_pb[1] + NU_, edge_pc[1] + (NU_ + NB_),
        edge_ps[1] + (NU_ + NB_ + NC_), edge_up[0], padt])
    sr = jnp.concatenate([srcA, srcB]).astype(i32).reshape(2, 16, NMAC, 2, 128)
    dr = jnp.concatenate([dstA, dstB]).astype(i32).reshape(2, 16, NMAC, 2, 128)
    return jnp.stack([sr, dr], axis=3).reshape(NM_ALL, 2, 2, 128)


def kernel(x_product, edge_pb, edge_pc, edge_ps, edge_up, user_emb,
           brand_emb, category_emb, shop_emb, Wp, bp, W1l, W1r, b1,
           W2l, W2r, b2):
    eidx = _edges(edge_pb, edge_pc, edge_ps, edge_up)
    zd = jnp.zeros((2, 128, 32), jnp.float32)
    zd16 = jnp.zeros((2, 128, 16), jnp.float32)
    deg16 = _sc_degree(eidx, zd16)

    prod = _tc_prod(x_product, Wp, bp)
    x = jnp.concatenate([prod, user_emb, brand_emb, category_emb, shop_emb],
                        axis=0)
    xlA, xlB, xrb = _tc_lin1(x, W1l, W1r, b1)

    aggA = _segsum(xlA, eidx, zd)
    aggB = _segsum(xlB, eidx, zd)
    hl, hrb2 = _tc_layer2in(aggA, aggB, deg16, xrb, W2l, W2r, b2)
    agg2 = _segsum(hl, eidx, zd)
    out = _tc_final(agg2, deg16, hrb2)
    return (out[:NP_], out[NP_:NP_ + NU_], out[NP_ + NU_:NP_ + NU_ + NB_],
            out[NP_ + NU_ + NB_:NP_ + NU_ + NB_ + NC_],
            out[NP_ + NU_ + NB_ + NC_:])
